# baseline (speedup 1.0000x reference)
import functools

import jax
import jax.numpy as jnp
from jax import lax
from jax.experimental import pallas as pl
from jax.experimental.pallas import tpu as pltpu
from jax.experimental.pallas import tpu_sc as plsc

R, C = 128, 32768
NC, NS, L = 2, 16, 16
NW = NC * NS
RPW = R // NW
NCHUNK = C // L
A = 8
GROUPS = NCHUNK // A

_BIG = 2**30


def _merge(b0, i0, b1, i1):
    take = (b1 > b0) | ((b1 == b0) & (i1 < i0))
    return jnp.where(take, b1, b0), jnp.where(take, i1, i0)


def _row_argmax(buf, vscr, iscr):
    iota = lax.iota(jnp.int32, L)
    best = tuple(buf[pl.ds(a * L, L)] for a in range(A))
    bidx = tuple(iota + a * L for a in range(A))
    cand = tuple(iota + (A + a) * L for a in range(A))

    def body(g, carry):
        best, bidx, cand = (list(t) for t in carry)
        base = g * (A * L)
        for a in range(A):
            x = buf[pl.ds(base + a * L, L)]
            m = x > best[a]
            best[a] = jnp.where(m, x, best[a])
            bidx[a] = jnp.where(m, cand[a], bidx[a])
            cand[a] = cand[a] + (A * L)
        return tuple(best), tuple(bidx), tuple(cand)

    best, bidx, _ = lax.fori_loop(1, GROUPS, body, (best, bidx, cand))

    best, bidx = list(best), list(bidx)
    step = 1
    while step < A:
        for a in range(0, A, 2 * step):
            best[a], bidx[a] = _merge(best[a], bidx[a],
                                      best[a + step], bidx[a + step])
        step *= 2

    b, i = best[0], bidx[0]
    for s in (1, 2, 4, 8):
        vscr[...] = b
        iscr[...] = i
        perm = iota ^ s
        b_p = plsc.load_gather(vscr, [perm])
        i_p = plsc.load_gather(iscr, [perm])
        b, i = _merge(b, i, b_p, i_p)
    return i


_mesh = plsc.VectorSubcoreMesh(
    core_axis_name="c", subcore_axis_name="s", num_cores=NC, num_subcores=NS)


@functools.partial(
    pl.kernel,
    out_type=jax.ShapeDtypeStruct((NW, L), jnp.int32),
    mesh=_mesh,
    scratch_types=[
        pltpu.VMEM((C,), jnp.float32),
        pltpu.VMEM((C,), jnp.float32),
        pltpu.VMEM((L,), jnp.int32),
        pltpu.VMEM((L,), jnp.float32),
        pltpu.VMEM((L,), jnp.int32),
        pltpu.SemaphoreType.DMA,
        pltpu.SemaphoreType.DMA,
    ],
    compiler_params=pltpu.CompilerParams(needs_layout_passes=False),
)
def _argmax_sc(x_hbm, out_hbm, buf0, buf1, res_v, vscr, iscr, sem0, sem1):
    wid = lax.axis_index("s") * NC + lax.axis_index("c")
    row0 = wid * RPW
    bufs = (buf0, buf1)
    sems = (sem0, sem1)
    copies = [None] * RPW
    copies[0] = pltpu.async_copy(
        x_hbm.at[pl.ds(row0 * C, C)], bufs[0], sems[0])
    lane = lax.iota(jnp.int32, L)
    res = jnp.zeros((L,), jnp.int32)
    for j in range(RPW):
        if j + 1 < RPW:
            copies[j + 1] = pltpu.async_copy(
                x_hbm.at[pl.ds((row0 + j + 1) * C, C)],
                bufs[(j + 1) % 2], sems[(j + 1) % 2])
        copies[j].wait()
        ridx = _row_argmax(bufs[j % 2], vscr, iscr)
        res = jnp.where(lane == j, ridx, res)
    res_v[...] = res
    pltpu.sync_copy(res_v, out_hbm.at[wid])


def kernel(tensor):
    out = _argmax_sc(tensor.reshape(R * C))
    return out[:, :RPW].reshape(R, 1)


# speedup vs baseline: 1.0973x; 1.0973x over previous
import functools

import jax
import jax.numpy as jnp
from jax import lax
from jax.experimental import pallas as pl
from jax.experimental.pallas import tpu as pltpu
from jax.experimental.pallas import tpu_sc as plsc

R, C = 128, 32768
NC, NS, L = 2, 16, 16
NW = NC * NS
TI, TJ = R // 8, C // 128
HJ = TJ // 2
NJ = 32
NSEG = HJ // NJ


def _merge(b0, i0, b1, i1):
    take = (b1 > b0) | ((b1 == b0) & (i1 < i0))
    return jnp.where(take, b1, b0), jnp.where(take, i1, i0)


def _butterfly(b, i, iota, vscr, iscr):
    for s in (1, 2, 4, 8):
        vscr[...] = b
        iscr[...] = i
        perm = iota ^ s
        b_p = plsc.load_gather(vscr, [perm])
        i_p = plsc.load_gather(iscr, [perm])
        b, i = _merge(b, i, b_p, i_p)
    return b, i


_mesh = plsc.VectorSubcoreMesh(
    core_axis_name="c", subcore_axis_name="s", num_cores=NC, num_subcores=NS)


@functools.partial(
    pl.kernel,
    out_type=(jax.ShapeDtypeStruct((NW, L), jnp.float32),
              jax.ShapeDtypeStruct((NW, L), jnp.int32)),
    mesh=_mesh,
    scratch_types=[
        pltpu.VMEM((NJ * 1024,), jnp.float32),
        pltpu.VMEM((NJ * 1024,), jnp.float32),
        pltpu.VMEM((L,), jnp.float32),
        pltpu.VMEM((L,), jnp.int32),
        pltpu.VMEM((L,), jnp.float32),
        pltpu.VMEM((L,), jnp.int32),
        pltpu.SemaphoreType.DMA,
        pltpu.SemaphoreType.DMA,
    ],
    compiler_params=pltpu.CompilerParams(needs_layout_passes=False),
)
def _argmax_sc(x_hbm, outv_hbm, outi_hbm, buf0, buf1, vscr, iscr,
               vals_v, idxs_v, sem0, sem1):
    c = lax.axis_index("c")
    s = lax.axis_index("s")
    wid = c * NS + s
    t = c * 8 + lax.rem(s, 8)
    h = lax.div(s, 8)
    base = t * (TJ * 1024) + h * (HJ * 1024)

    bufs = (buf0, buf1)
    sems = (sem0, sem1)
    copies = [None] * NSEG
    copies[0] = pltpu.async_copy(
        x_hbm.at[pl.ds(base, NJ * 1024)], bufs[0], sems[0])

    iota = lax.iota(jnp.int32, L)
    ninf = jnp.full((L,), -jnp.inf, jnp.float32)
    best = [ninf for _ in range(8)]
    bidx = [jnp.zeros((L,), jnp.int32) for _ in range(8)]
    cand = [h * (HJ * 128) + cc * 16 + iota for cc in range(8)]

    def body(jj, carry, *, buf):
        best, bidx, cand = (list(x) for x in carry)
        off = jj * 1024
        for kk in range(8):
            for cc in range(8):
                x = buf[pl.ds(off + kk * 128 + cc * 16, 16)]
                m = x > best[kk]
                best[kk] = jnp.where(m, x, best[kk])
                bidx[kk] = jnp.where(m, cand[cc], bidx[kk])
        for cc in range(8):
            cand[cc] = cand[cc] + 128
        return tuple(best), tuple(bidx), tuple(cand)

    carry = (tuple(best), tuple(bidx), tuple(cand))
    for seg in range(NSEG):
        if seg + 1 < NSEG:
            copies[seg + 1] = pltpu.async_copy(
                x_hbm.at[pl.ds(base + (seg + 1) * NJ * 1024, NJ * 1024)],
                bufs[(seg + 1) % 2], sems[(seg + 1) % 2])
        copies[seg].wait()
        carry = lax.fori_loop(
            0, NJ, functools.partial(body, buf=bufs[seg % 2]), carry)
    best, bidx, _ = carry

    vals = jnp.zeros((L,), jnp.float32)
    idxs = jnp.zeros((L,), jnp.int32)
    for kk in range(8):
        b, i = _butterfly(best[kk], bidx[kk], iota, vscr, iscr)
        sel = iota == kk
        vals = jnp.where(sel, b, vals)
        idxs = jnp.where(sel, i, idxs)

    vals_v[...] = vals
    idxs_v[...] = idxs
    pltpu.sync_copy(vals_v, outv_hbm.at[wid])
    pltpu.sync_copy(idxs_v, outi_hbm.at[wid])


def kernel(tensor):
    x1 = (tensor.reshape(TI, 8, TJ, 128)
          .transpose(0, 2, 1, 3)
          .reshape(TI * TJ * 1024))
    outv, outi = _argmax_sc(x1)
    v = outv.reshape(NC, 2, 8, L)[:, :, :, :8]
    i = outi.reshape(NC, 2, 8, L)[:, :, :, :8]
    take = v[:, 1] > v[:, 0]
    idx = jnp.where(take, i[:, 1], i[:, 0])
    return idx.reshape(R, 1)


# speedup vs baseline: 1.3859x; 1.2630x over previous
import functools

import jax
import jax.numpy as jnp
from jax import lax
from jax.experimental import pallas as pl
from jax.experimental.pallas import tpu as pltpu
from jax.experimental.pallas import tpu_sc as plsc

R, C = 128, 32768
NC, NS, L = 2, 16, 16
NW = NC * NS
TI, TJ = R // 8, C // 128
HJ = TJ // 2
NJ = 16
NSEG = HJ // NJ


def _merge(b0, i0, b1, i1):
    take = (b1 > b0) | ((b1 == b0) & (i1 < i0))
    return jnp.where(take, b1, b0), jnp.where(take, i1, i0)


def _butterfly(b, i, iota, vscr, iscr):
    for s in (1, 2, 4, 8):
        vscr[...] = b
        iscr[...] = i
        perm = iota ^ s
        b_p = plsc.load_gather(vscr, [perm])
        i_p = plsc.load_gather(iscr, [perm])
        b, i = _merge(b, i, b_p, i_p)
    return b, i


_mesh = plsc.VectorSubcoreMesh(
    core_axis_name="c", subcore_axis_name="s", num_cores=NC, num_subcores=NS)


@functools.partial(
    pl.kernel,
    out_type=(jax.ShapeDtypeStruct((NW, L), jnp.float32),
              jax.ShapeDtypeStruct((NW, L), jnp.int32)),
    mesh=_mesh,
    scratch_types=[
        pltpu.VMEM((NJ * 1024,), jnp.float32),
        pltpu.VMEM((NJ * 1024,), jnp.float32),
        pltpu.VMEM((L,), jnp.float32),
        pltpu.VMEM((L,), jnp.int32),
        pltpu.VMEM((L,), jnp.float32),
        pltpu.VMEM((L,), jnp.int32),
        pltpu.SemaphoreType.DMA,
        pltpu.SemaphoreType.DMA,
    ],
    compiler_params=pltpu.CompilerParams(needs_layout_passes=False),
)
def _argmax_sc(x_hbm, outv_hbm, outi_hbm, buf0, buf1, vscr, iscr,
               vals_v, idxs_v, sem0, sem1):
    c = lax.axis_index("c")
    s = lax.axis_index("s")
    wid = c * NS + s
    t = c * 8 + lax.rem(s, 8)
    h = lax.div(s, 8)
    base = t * (TJ * 1024) + h * (HJ * 1024)

    bufs = (buf0, buf1)
    sems = (sem0, sem1)
    copies = [None] * NSEG
    copies[0] = pltpu.async_copy(
        x_hbm.at[pl.ds(base, NJ * 1024)], bufs[0], sems[0])

    iota = lax.iota(jnp.int32, L)
    ninf = jnp.full((L,), -jnp.inf, jnp.float32)
    acc = {(kk, p): (ninf, jnp.zeros((L,), jnp.int32))
           for kk in range(8) for p in range(2)}

    def make_body(buf, seg_tile_base, kks):
        def body(jj, carry):
            a = {(kk, p): list(carry[4 * ki + 2 * p: 4 * ki + 2 * p + 2])
                 for ki, kk in enumerate(kks) for p in range(2)}
            off = jj * 1024
            cid_base = (seg_tile_base + jj) * 8
            for kk in kks:
                for cc in range(8):
                    p = cc & 1
                    b, i = a[(kk, p)]
                    x = buf[pl.ds(off + kk * 128 + cc * 16, 16)]
                    m = x > b
                    cid = jnp.full((L,), cid_base + cc, jnp.int32)
                    a[(kk, p)] = [jnp.where(m, x, b), jnp.where(m, cid, i)]
            out = []
            for ki, kk in enumerate(kks):
                for p in range(2):
                    out.extend(a[(kk, p)])
            return tuple(out)
        return body

    for seg in range(NSEG):
        if seg + 1 < NSEG:
            copies[seg + 1] = pltpu.async_copy(
                x_hbm.at[pl.ds(base + (seg + 1) * NJ * 1024, NJ * 1024)],
                bufs[(seg + 1) % 2], sems[(seg + 1) % 2])
        copies[seg].wait()
        buf = bufs[seg % 2]
        stb = h * HJ + seg * NJ
        for kks in ((0, 1, 2, 3), (4, 5, 6, 7)):
            carry = tuple(v for kk in kks for p in range(2)
                          for v in acc[(kk, p)])
            carry = lax.fori_loop(0, NJ, make_body(buf, stb, kks), carry)
            for ki, kk in enumerate(kks):
                for p in range(2):
                    acc[(kk, p)] = list(carry[4 * ki + 2 * p: 4 * ki + 2 * p + 2])

    best, bidx = [], []
    for kk in range(8):
        b0, i0 = acc[(kk, 0)]
        b1, i1 = acc[(kk, 1)]
        i0 = i0 * 16 + iota
        i1 = i1 * 16 + iota
        b, i = _merge(b0, i0, b1, i1)
        best.append(b)
        bidx.append(i)

    vals = jnp.zeros((L,), jnp.float32)
    idxs = jnp.zeros((L,), jnp.int32)
    for kk in range(8):
        b, i = _butterfly(best[kk], bidx[kk], iota, vscr, iscr)
        sel = iota == kk
        vals = jnp.where(sel, b, vals)
        idxs = jnp.where(sel, i, idxs)

    vals_v[...] = vals
    idxs_v[...] = idxs
    pltpu.sync_copy(vals_v, outv_hbm.at[wid])
    pltpu.sync_copy(idxs_v, outi_hbm.at[wid])


def kernel(tensor):
    x1 = (tensor.reshape(TI, 8, TJ, 128)
          .transpose(0, 2, 1, 3)
          .reshape(TI * TJ * 1024))
    outv, outi = _argmax_sc(x1)
    v = outv.reshape(NC, 2, 8, L)[:, :, :, :8]
    i = outi.reshape(NC, 2, 8, L)[:, :, :, :8]
    take = v[:, 1] > v[:, 0]
    idx = jnp.where(take, i[:, 1], i[:, 0])
    return idx.reshape(R, 1)
